# baseline (speedup 1.0000x reference)
import functools

import jax
import jax.numpy as jnp
from jax import lax
from jax.experimental import pallas as pl
from jax.experimental.pallas import tpu as pltpu
from jax.experimental.pallas import tpu_sc as plsc


def _tc_pair_pack(table_t, half):
    d, v = table_t.shape
    n_lane_blocks = half // 128
    k = 17
    steps = n_lane_blocks // k
    blk = 128 * k
    v_pad = 2 * half

    def body(in_ref, out_ref):
        j = pl.program_id(0)
        left = jnp.transpose(in_ref[:, pl.ds(j * blk, blk)])
        right = jnp.transpose(in_ref[:, pl.ds(half + j * blk, blk)])
        out_ref[...] = jnp.concatenate([left, right], axis=1)

    return pl.pallas_call(
        body,
        grid=(steps,),
        in_specs=[
            pl.BlockSpec((d, v_pad), lambda j: (0, 0)),
        ],
        out_specs=pl.BlockSpec((blk, 2 * d), lambda j: (j, 0)),
        out_shape=jax.ShapeDtypeStruct((half, 2 * d), jnp.float32),
    )(table_t)


def _sc_gather(table, idx):
    v, d = table.shape
    b = idx.shape[0]
    nc, ns = 2, 16
    nw = nc * ns
    b_per_w = b // nw
    ch = 128
    n_chunks = b_per_w // ch
    mesh = plsc.VectorSubcoreMesh(core_axis_name="c", subcore_axis_name="s")

    @functools.partial(
        pl.kernel,
        mesh=mesh,
        out_type=jax.ShapeDtypeStruct((b, d), table.dtype),
        scratch_types=[
            pltpu.VMEM((b_per_w,), jnp.int32),
            pltpu.VMEM((ch, d), table.dtype),
            pltpu.SemaphoreType.DMA,
        ],
    )
    def k(table_hbm, idx_hbm, out_hbm, idx_v, buf, sem):
        wid = lax.axis_index("s") * nc + lax.axis_index("c")
        base = wid * b_per_w
        pltpu.sync_copy(idx_hbm.at[pl.ds(base, b_per_w)], idx_v)
        for j in range(n_chunks):
            pltpu.async_copy(
                table_hbm.at[idx_v.at[pl.ds(j * ch, ch)]], buf, sem
            ).wait()
            pltpu.sync_copy(buf, out_hbm.at[pl.ds(base + j * ch, ch)])

    return k(table, idx)


def _tc_project_chunk(rows2, ids3, proj_w, scale_arr, half, b_total,
                      chunk_idx, n_chunks, prev_out):
    bc = rows2.shape[0]
    m, d = proj_w.shape
    mb = 1024
    grid = bc // mb
    base = chunk_idx * grid

    def body(*refs):
        if prev_out is None:
            rows_ref, ids_ref, w_ref, scale_ref, out_ref = refs
        else:
            _, rows_ref, ids_ref, w_ref, scale_ref, out_ref = refs
        sel = jnp.reshape(ids_ref[0, 0, :], (mb, 1))
        rows = rows_ref[...]
        h = jnp.where(sel == 1, rows[:, d:], rows[:, :d]).astype(jnp.bfloat16)
        acc = lax.dot_general(
            h,
            w_ref[...].astype(jnp.bfloat16),
            dimension_numbers=(((1,), (1,)), ((), ())),
            preferred_element_type=jnp.float32,
        )
        out_ref[...] = acc * scale_ref[0]

    in_specs = [
        pl.BlockSpec((mb, 2 * d), lambda i: (i, 0)),
        pl.BlockSpec((1, 1, mb), lambda i: (i, 0, 0)),
        pl.BlockSpec((m, d), lambda i: (0, 0)),
        pl.BlockSpec(memory_space=pltpu.SMEM),
    ]
    args = (rows2, ids3, proj_w, scale_arr)
    aliases = {}
    if prev_out is not None:
        in_specs = [pl.BlockSpec(memory_space=pltpu.HBM)] + in_specs
        args = (prev_out,) + args
        aliases = {0: 0}
    return pl.pallas_call(
        body,
        grid=(grid,),
        in_specs=in_specs,
        out_specs=pl.BlockSpec((mb, m), lambda i: (base + i, 0)),
        out_shape=jax.ShapeDtypeStruct((b_total, m), jnp.float32),
        input_output_aliases=aliases,
    )(*args)


def kernel(token_ids, embed_weight, proj_weight, scale):
    batch, seq = token_ids.shape
    v, d = embed_weight.shape
    half = 50048
    model_dim = proj_weight.shape[0]
    ids = token_ids.reshape(-1).astype(jnp.int32)
    table_t = jnp.swapaxes(embed_weight, 0, 1)
    pairs = _tc_pair_pack(table_t, half)
    sel = (ids >= half).astype(jnp.int32)
    idx = ids - sel * half
    scale_arr = jnp.reshape(scale, (1,)).astype(jnp.float32)
    b_total = ids.shape[0]
    n_chunks = 2
    bc = b_total // n_chunks
    out = None
    for c in range(n_chunks):
        rows_c = _sc_gather(pairs, idx[c * bc:(c + 1) * bc])
        sel3_c = sel[c * bc:(c + 1) * bc].reshape(-1, 1, 1024)
        out = _tc_project_chunk(rows_c, sel3_c, proj_weight, scale_arr, half,
                                b_total, c, n_chunks, out)
    return out.reshape(batch, seq, model_dim)


# speedup vs baseline: 1.0755x; 1.0755x over previous
import functools

import jax
import jax.numpy as jnp
from jax import lax
from jax.experimental import pallas as pl
from jax.experimental.pallas import tpu as pltpu
from jax.experimental.pallas import tpu_sc as plsc


def _tc_pair_pack(table_t, half):
    d, v = table_t.shape
    n_lane_blocks = half // 128
    k = 17
    steps = n_lane_blocks // k
    blk = 128 * k
    v_pad = 2 * half

    def body(in_ref, out_ref):
        j = pl.program_id(0)
        left = jnp.transpose(in_ref[:, pl.ds(j * blk, blk)])
        right = jnp.transpose(in_ref[:, pl.ds(half + j * blk, blk)])
        out_ref[...] = jnp.concatenate([left, right], axis=1)

    return pl.pallas_call(
        body,
        grid=(steps,),
        in_specs=[
            pl.BlockSpec((d, v_pad), lambda j: (0, 0)),
        ],
        out_specs=pl.BlockSpec((blk, 2 * d), lambda j: (j, 0)),
        out_shape=jax.ShapeDtypeStruct((half, 2 * d), jnp.float32),
    )(table_t)


def _sc_gather(table, idx):
    v, d = table.shape
    b = idx.shape[0]
    nc, ns = 2, 16
    nw = nc * ns
    b_per_w = b // nw
    ch = 128
    n_chunks = b_per_w // ch
    mesh = plsc.VectorSubcoreMesh(core_axis_name="c", subcore_axis_name="s")

    @functools.partial(
        pl.kernel,
        mesh=mesh,
        out_type=jax.ShapeDtypeStruct((b, d), table.dtype),
        scratch_types=[
            pltpu.VMEM((b_per_w,), jnp.int32),
            pltpu.VMEM((b_per_w, d), table.dtype),
            pltpu.SemaphoreType.DMA,
        ],
    )
    def k(table_hbm, idx_hbm, out_hbm, idx_v, buf, sem):
        wid = lax.axis_index("s") * nc + lax.axis_index("c")
        base = wid * b_per_w
        pltpu.sync_copy(idx_hbm.at[pl.ds(base, b_per_w)], idx_v)
        for j in range(n_chunks):
            pltpu.async_copy(
                table_hbm.at[idx_v.at[pl.ds(j * ch, ch)]],
                buf.at[pl.ds(j * ch, ch)],
                sem,
            )
        for j in range(n_chunks):
            pltpu.make_async_copy(
                table_hbm.at[idx_v.at[pl.ds(j * ch, ch)]],
                buf.at[pl.ds(j * ch, ch)],
                sem,
            ).wait()
        pltpu.sync_copy(buf, out_hbm.at[pl.ds(base, b_per_w)])

    return k(table, idx)


def _tc_project_chunk(rows2, ids3, proj_w, scale_arr, half, b_total,
                      chunk_idx, n_chunks, prev_out):
    bc = rows2.shape[0]
    m, d = proj_w.shape
    mb = 2048
    grid = bc // mb
    base = chunk_idx * grid

    def body(*refs):
        if prev_out is None:
            rows_ref, ids_ref, w_ref, scale_ref, out_ref = refs
        else:
            _, rows_ref, ids_ref, w_ref, scale_ref, out_ref = refs
        sel = jnp.reshape(ids_ref[0, 0, :], (mb, 1))
        rows = rows_ref[...]
        h = jnp.where(sel == 1, rows[:, d:], rows[:, :d]).astype(jnp.bfloat16)
        acc = lax.dot_general(
            h,
            w_ref[...].astype(jnp.bfloat16),
            dimension_numbers=(((1,), (1,)), ((), ())),
            preferred_element_type=jnp.float32,
        )
        out_ref[...] = acc * scale_ref[0]

    in_specs = [
        pl.BlockSpec((mb, 2 * d), lambda i: (i, 0)),
        pl.BlockSpec((1, 1, mb), lambda i: (i, 0, 0)),
        pl.BlockSpec((m, d), lambda i: (0, 0)),
        pl.BlockSpec(memory_space=pltpu.SMEM),
    ]
    args = (rows2, ids3, proj_w, scale_arr)
    aliases = {}
    if prev_out is not None:
        in_specs = [pl.BlockSpec(memory_space=pltpu.HBM)] + in_specs
        args = (prev_out,) + args
        aliases = {0: 0}
    return pl.pallas_call(
        body,
        grid=(grid,),
        in_specs=in_specs,
        out_specs=pl.BlockSpec((mb, m), lambda i: (base + i, 0)),
        out_shape=jax.ShapeDtypeStruct((b_total, m), jnp.float32),
        input_output_aliases=aliases,
    )(*args)


def kernel(token_ids, embed_weight, proj_weight, scale):
    batch, seq = token_ids.shape
    v, d = embed_weight.shape
    half = 50048
    model_dim = proj_weight.shape[0]
    ids = token_ids.reshape(-1).astype(jnp.int32)
    table_t = jnp.swapaxes(embed_weight, 0, 1)
    pairs = _tc_pair_pack(table_t, half)
    sel = (ids >= half).astype(jnp.int32)
    idx = ids - sel * half
    scale_arr = jnp.reshape(scale, (1,)).astype(jnp.float32)
    b_total = ids.shape[0]
    rows2 = _sc_gather(pairs, idx)
    sel3 = sel.reshape(-1, 1, 2048)
    out = _tc_project_chunk(rows2, sel3, proj_weight, scale_arr, half,
                            b_total, 0, 1, None)
    return out.reshape(batch, seq, model_dim)
